# initial kernel scaffold (unmeasured)
import jax
import jax.numpy as jnp
from jax import lax
from jax.experimental import pallas as pl
from jax.experimental.pallas import tpu as pltpu

N_DEV = 32
N_R = 16
N_L = 15


def kernel(x, w_mat, scale_x, scale_w):
    m_per, k = x.shape
    _, n_per = w_mat.shape

    def body(x_ref, w_ref, sx_ref, sw_ref, out_ref,
             rbuf, lbuf, send_r, send_l, recv_r, recv_l):
        my = lax.axis_index("i")
        right = lax.rem(my + 1, N_DEV)
        left = lax.rem(my + (N_DEV - 1), N_DEV)

        barrier = pltpu.get_barrier_semaphore()
        for nbr in (left, right):
            pl.semaphore_signal(barrier, inc=1, device_id=(nbr,),
                                device_id_type=pl.DeviceIdType.MESH)
        pl.semaphore_wait(barrier, 2)

        scale = sx_ref[0, 0] * sw_ref[0, 0]

        def compute(chunk_ref, origin):
            acc = jnp.dot(chunk_ref[...], w_ref[...],
                          preferred_element_type=jnp.float32)
            y = jnp.maximum(acc * scale, 0.0)
            out_ref[pl.ds(origin * m_per, m_per), :] = y

        def mk(src, dst_buf, slot, ssem, rsem, dev):
            return pltpu.make_async_remote_copy(
                src_ref=src, dst_ref=dst_buf.at[slot],
                send_sem=ssem, recv_sem=rsem.at[slot],
                device_id=(dev,), device_id_type=pl.DeviceIdType.MESH)

        r = mk(x_ref, rbuf, 0, send_r.at[0], recv_r, right)
        l = mk(x_ref, lbuf, 0, send_l.at[0], recv_l, left)
        r.start()
        l.start()
        compute(x_ref, my)
        r.wait()
        compute(rbuf.at[0], lax.rem(my + (N_DEV - 1), N_DEV))
        l.wait()
        compute(lbuf.at[0], lax.rem(my + 1, N_DEV))

        for h in range(1, N_R):
            r = mk(rbuf.at[h - 1], rbuf, h, send_r.at[h], recv_r, right)
            r.start()
            if h < N_L:
                l = mk(lbuf.at[h - 1], lbuf, h, send_l.at[h], recv_l, left)
                l.start()
            r.wait()
            compute(rbuf.at[h], lax.rem(my + (N_DEV - 1 - h), N_DEV))
            if h < N_L:
                l.wait()
                compute(lbuf.at[h], lax.rem(my + (1 + h), N_DEV))

    out_shape = jax.ShapeDtypeStruct((N_DEV * m_per, n_per), jnp.float32)
    return pl.pallas_call(
        body,
        out_shape=out_shape,
        in_specs=[
            pl.BlockSpec(memory_space=pltpu.VMEM),
            pl.BlockSpec(memory_space=pltpu.VMEM),
            pl.BlockSpec(memory_space=pltpu.SMEM),
            pl.BlockSpec(memory_space=pltpu.SMEM),
        ],
        out_specs=pl.BlockSpec(memory_space=pltpu.VMEM),
        scratch_shapes=[
            pltpu.VMEM((N_R, m_per, k), x.dtype),
            pltpu.VMEM((N_L, m_per, k), x.dtype),
            pltpu.SemaphoreType.DMA((N_R,)),
            pltpu.SemaphoreType.DMA((N_L,)),
            pltpu.SemaphoreType.DMA((N_R,)),
            pltpu.SemaphoreType.DMA((N_L,)),
        ],
        compiler_params=pltpu.CompilerParams(collective_id=0),
    )(x, w_mat, scale_x.reshape(1, 1), scale_w.reshape(1, 1))


# baseline (device time: 219947 ns/iter reference)
import jax
import jax.numpy as jnp
from jax import lax
from jax.experimental import pallas as pl
from jax.experimental.pallas import tpu as pltpu

N_DEV = 32
N_R = 16
N_L = 15


def kernel(x, w_mat, scale_x, scale_w):
    m_per, k = x.shape
    _, n_per = w_mat.shape

    def body(x_ref, w_ref, sx_ref, sw_ref, out_ref,
             xq, wq, rbuf, lbuf, send_r, send_l, recv_r, recv_l):
        my = lax.axis_index("i")
        right = lax.rem(my + 1, N_DEV)
        left = lax.rem(my + (N_DEV - 1), N_DEV)

        xq[...] = x_ref[...].astype(jnp.float8_e5m2)
        wq[...] = w_ref[...].astype(jnp.float8_e5m2)

        barrier = pltpu.get_barrier_semaphore()
        for nbr in (left, right):
            pl.semaphore_signal(barrier, inc=1, device_id=(nbr,),
                                device_id_type=pl.DeviceIdType.MESH)
        pl.semaphore_wait(barrier, 2)

        scale = sx_ref[0, 0] * sw_ref[0, 0]

        def compute(chunk_ref, origin):
            acc = jnp.dot(chunk_ref[...], wq[...],
                          preferred_element_type=jnp.float32)
            y = jnp.maximum(acc * scale, 0.0)
            out_ref[pl.ds(origin * m_per, m_per), :] = y

        def mk(src, dst_buf, slot, ssem, rsem, dev):
            return pltpu.make_async_remote_copy(
                src_ref=src, dst_ref=dst_buf.at[slot],
                send_sem=ssem, recv_sem=rsem.at[slot],
                device_id=(dev,), device_id_type=pl.DeviceIdType.MESH)

        r = mk(xq, rbuf, 0, send_r.at[0], recv_r, right)
        l = mk(xq, lbuf, 0, send_l.at[0], recv_l, left)
        r.start()
        l.start()
        compute(xq, my)
        r.wait()
        compute(rbuf.at[0], lax.rem(my + (N_DEV - 1), N_DEV))
        l.wait()
        compute(lbuf.at[0], lax.rem(my + 1, N_DEV))

        for h in range(1, N_R):
            r = mk(rbuf.at[h - 1], rbuf, h, send_r.at[h], recv_r, right)
            r.start()
            if h < N_L:
                l = mk(lbuf.at[h - 1], lbuf, h, send_l.at[h], recv_l, left)
                l.start()
            r.wait()
            compute(rbuf.at[h], lax.rem(my + (N_DEV - 1 - h), N_DEV))
            if h < N_L:
                l.wait()
                compute(lbuf.at[h], lax.rem(my + (1 + h), N_DEV))

    out_shape = jax.ShapeDtypeStruct((N_DEV * m_per, n_per), jnp.float32)
    return pl.pallas_call(
        body,
        out_shape=out_shape,
        in_specs=[
            pl.BlockSpec(memory_space=pltpu.VMEM),
            pl.BlockSpec(memory_space=pltpu.VMEM),
            pl.BlockSpec(memory_space=pltpu.SMEM),
            pl.BlockSpec(memory_space=pltpu.SMEM),
        ],
        out_specs=pl.BlockSpec(memory_space=pltpu.VMEM),
        scratch_shapes=[
            pltpu.VMEM((m_per, k), jnp.float8_e5m2),
            pltpu.VMEM((k, n_per), jnp.float8_e5m2),
            pltpu.VMEM((N_R, m_per, k), jnp.float8_e5m2),
            pltpu.VMEM((N_L, m_per, k), jnp.float8_e5m2),
            pltpu.SemaphoreType.DMA((N_R,)),
            pltpu.SemaphoreType.DMA((N_L,)),
            pltpu.SemaphoreType.DMA((N_R,)),
            pltpu.SemaphoreType.DMA((N_L,)),
        ],
        compiler_params=pltpu.CompilerParams(collective_id=0, vmem_limit_bytes=64 * 1024 * 1024),
    )(x, w_mat, scale_x.reshape(1, 1), scale_w.reshape(1, 1))


# device time: 193323 ns/iter; 1.1377x vs baseline; 1.1377x over previous
import jax
import jax.numpy as jnp
from jax import lax
from jax.experimental import pallas as pl
from jax.experimental.pallas import tpu as pltpu

N_DEV = 32
N_R = 16
N_L = 15
SUBS = 2


def kernel(x, w_mat, scale_x, scale_w):
    m_per, k = x.shape
    _, n_per = w_mat.shape
    m_sub = m_per // SUBS

    def body(x_ref, w_ref, sx_ref, sw_ref, out_ref,
             xq, wq, rbuf, lbuf, send_r, send_l, recv_r, recv_l):
        my = lax.axis_index("i")
        right = lax.rem(my + 1, N_DEV)
        left = lax.rem(my + (N_DEV - 1), N_DEV)

        xq[...] = x_ref[...].astype(jnp.float8_e5m2)

        barrier = pltpu.get_barrier_semaphore()
        for nbr in (left, right):
            pl.semaphore_signal(barrier, inc=1, device_id=(nbr,),
                                device_id_type=pl.DeviceIdType.MESH)
        pl.semaphore_wait(barrier, 2)

        def sub(ref_2d, j):
            return ref_2d.at[pl.ds(j * m_sub, m_sub), :]

        def mk(src, buf, h, j, ssem, rsem, dev):
            return pltpu.make_async_remote_copy(
                src_ref=src, dst_ref=sub(buf.at[h], j),
                send_sem=ssem.at[h, j], recv_sem=rsem.at[h, j],
                device_id=(dev,), device_id_type=pl.DeviceIdType.MESH)

        started = []

        for j in range(SUBS):
            d = mk(sub(xq, j), rbuf, 0, j, send_r, recv_r, right)
            d.start()
            started.append(d)
            d = mk(sub(xq, j), lbuf, 0, j, send_l, recv_l, left)
            d.start()
            started.append(d)

        wq[...] = w_ref[...].astype(jnp.float8_e5m2)
        scale = sx_ref[0, 0] * sw_ref[0, 0]

        def compute(chunk_ref, origin):
            acc = jnp.dot(chunk_ref[...], wq[...],
                          preferred_element_type=jnp.float32)
            y = jnp.maximum(acc * scale, 0.0)
            out_ref[pl.ds(origin * m_per, m_per), :] = y

        compute(xq, my)

        for h in range(N_R):
            for j in range(SUBS):
                mk(sub(rbuf.at[h], j), rbuf, h, j,
                   send_r, recv_r, right).wait_recv()
                if h + 1 < N_R:
                    d = mk(sub(rbuf.at[h], j), rbuf, h + 1, j,
                           send_r, recv_r, right)
                    d.start()
                    started.append(d)
            if h < N_L:
                for j in range(SUBS):
                    mk(sub(lbuf.at[h], j), lbuf, h, j,
                       send_l, recv_l, left).wait_recv()
                    if h + 1 < N_L:
                        d = mk(sub(lbuf.at[h], j), lbuf, h + 1, j,
                               send_l, recv_l, left)
                        d.start()
                        started.append(d)
            compute(rbuf.at[h], lax.rem(my + (N_DEV - 1 - h), N_DEV))
            if h < N_L:
                compute(lbuf.at[h], lax.rem(my + (1 + h), N_DEV))

        for d in started:
            d.wait_send()

    out_shape = jax.ShapeDtypeStruct((N_DEV * m_per, n_per), jnp.float32)
    return pl.pallas_call(
        body,
        out_shape=out_shape,
        in_specs=[
            pl.BlockSpec(memory_space=pltpu.VMEM),
            pl.BlockSpec(memory_space=pltpu.VMEM),
            pl.BlockSpec(memory_space=pltpu.SMEM),
            pl.BlockSpec(memory_space=pltpu.SMEM),
        ],
        out_specs=pl.BlockSpec(memory_space=pltpu.VMEM),
        scratch_shapes=[
            pltpu.VMEM((m_per, k), jnp.float8_e5m2),
            pltpu.VMEM((k, n_per), jnp.float8_e5m2),
            pltpu.VMEM((N_R, m_per, k), jnp.float8_e5m2),
            pltpu.VMEM((N_L, m_per, k), jnp.float8_e5m2),
            pltpu.SemaphoreType.DMA((N_R, SUBS)),
            pltpu.SemaphoreType.DMA((N_L, SUBS)),
            pltpu.SemaphoreType.DMA((N_R, SUBS)),
            pltpu.SemaphoreType.DMA((N_L, SUBS)),
        ],
        compiler_params=pltpu.CompilerParams(
            collective_id=0, vmem_limit_bytes=64 * 1024 * 1024),
    )(x, w_mat, scale_x.reshape(1, 1), scale_w.reshape(1, 1))


# device time: 105322 ns/iter; 2.0883x vs baseline; 1.8355x over previous
import jax
import jax.numpy as jnp
from jax import lax
from jax.experimental import pallas as pl
from jax.experimental.pallas import tpu as pltpu

N_DEV = 32
N_R = 16
N_L = 15
SUBS = 2

P_RING = [0, 3, 4, 7, 15, 12, 11, 8, 16, 19, 20, 23, 31, 28, 27, 24,
          25, 26, 29, 30, 22, 21, 18, 17, 9, 10, 13, 14, 6, 5, 2, 1]
Q_RING = [0, 31, 30, 1, 2, 29, 28, 3, 7, 24, 25, 6, 5, 26, 27, 4,
          8, 23, 22, 9, 10, 21, 20, 11, 15, 16, 17, 14, 13, 18, 19, 12]


def kernel(x, w_mat, scale_x, scale_w):
    m_per, k = x.shape
    _, n_per = w_mat.shape
    m_sub = m_per // SUBS

    def body(x_ref, w_ref, sx_ref, sw_ref, p_ref, q_ref, out_ref,
             xq, wq, rbuf, lbuf, send_r, send_l, recv_r, recv_l):
        my = lax.axis_index("i")
        q = q_ref[my]
        right = p_ref[lax.rem(q + 1, N_DEV)]
        left = p_ref[lax.rem(q + (N_DEV - 1), N_DEV)]

        xq[...] = x_ref[...].astype(jnp.float8_e5m2)

        barrier = pltpu.get_barrier_semaphore()
        for nbr in (left, right):
            pl.semaphore_signal(barrier, inc=1, device_id=(nbr,),
                                device_id_type=pl.DeviceIdType.MESH)
        pl.semaphore_wait(barrier, 2)

        def sub(ref_2d, j):
            return ref_2d.at[pl.ds(j * m_sub, m_sub), :]

        def mk(src, buf, h, j, ssem, rsem, dev):
            return pltpu.make_async_remote_copy(
                src_ref=src, dst_ref=sub(buf.at[h], j),
                send_sem=ssem.at[h, j], recv_sem=rsem.at[h, j],
                device_id=(dev,), device_id_type=pl.DeviceIdType.MESH)

        started = []

        for j in range(SUBS):
            d = mk(sub(xq, j), rbuf, 0, j, send_r, recv_r, right)
            d.start()
            started.append(d)
            d = mk(sub(xq, j), lbuf, 0, j, send_l, recv_l, left)
            d.start()
            started.append(d)

        wq[...] = w_ref[...].astype(jnp.float8_e5m2)
        scale = sx_ref[0, 0] * sw_ref[0, 0]

        def compute(chunk_ref, origin):
            acc = jnp.dot(chunk_ref[...], wq[...],
                          preferred_element_type=jnp.float32)
            y = jnp.maximum(acc * scale, 0.0)
            out_ref[pl.ds(origin * m_per, m_per), :] = y

        compute(xq, my)

        for h in range(N_R):
            for j in range(SUBS):
                mk(sub(rbuf.at[h], j), rbuf, h, j,
                   send_r, recv_r, right).wait_recv()
                if h + 1 < N_R:
                    d = mk(sub(rbuf.at[h], j), rbuf, h + 1, j,
                           send_r, recv_r, right)
                    d.start()
                    started.append(d)
            if h < N_L:
                for j in range(SUBS):
                    mk(sub(lbuf.at[h], j), lbuf, h, j,
                       send_l, recv_l, left).wait_recv()
                    if h + 1 < N_L:
                        d = mk(sub(lbuf.at[h], j), lbuf, h + 1, j,
                               send_l, recv_l, left)
                        d.start()
                        started.append(d)
            compute(rbuf.at[h], p_ref[lax.rem(q + (N_DEV - 1 - h), N_DEV)])
            if h < N_L:
                compute(lbuf.at[h], p_ref[lax.rem(q + (1 + h), N_DEV)])

        for d in started:
            d.wait_send()

    out_shape = jax.ShapeDtypeStruct((N_DEV * m_per, n_per), jnp.float32)
    return pl.pallas_call(
        body,
        out_shape=out_shape,
        in_specs=[
            pl.BlockSpec(memory_space=pltpu.VMEM),
            pl.BlockSpec(memory_space=pltpu.VMEM),
            pl.BlockSpec(memory_space=pltpu.SMEM),
            pl.BlockSpec(memory_space=pltpu.SMEM),
            pl.BlockSpec(memory_space=pltpu.SMEM),
            pl.BlockSpec(memory_space=pltpu.SMEM),
        ],
        out_specs=pl.BlockSpec(memory_space=pltpu.VMEM),
        scratch_shapes=[
            pltpu.VMEM((m_per, k), jnp.float8_e5m2),
            pltpu.VMEM((k, n_per), jnp.float8_e5m2),
            pltpu.VMEM((N_R, m_per, k), jnp.float8_e5m2),
            pltpu.VMEM((N_L, m_per, k), jnp.float8_e5m2),
            pltpu.SemaphoreType.DMA((N_R, SUBS)),
            pltpu.SemaphoreType.DMA((N_L, SUBS)),
            pltpu.SemaphoreType.DMA((N_R, SUBS)),
            pltpu.SemaphoreType.DMA((N_L, SUBS)),
        ],
        compiler_params=pltpu.CompilerParams(
            collective_id=0, vmem_limit_bytes=64 * 1024 * 1024),
    )(x, w_mat, scale_x.reshape(1, 1), scale_w.reshape(1, 1),
      jnp.asarray(P_RING, jnp.int32), jnp.asarray(Q_RING, jnp.int32))
